# Initial kernel scaffold; baseline (speedup 1.0000x reference)
import jax
import jax.numpy as jnp
from jax.experimental import pallas as pl


def kernel(pc, feature, params):
    raise NotImplementedError("write your pallas kernel here")


# speedup vs baseline: 1.0004x; 1.0004x over previous
import jax
import jax.numpy as jnp
from jax.experimental import pallas as pl

_NPOINT = 8192


def _sqdist(src, dst):
    return (jnp.sum(src ** 2, axis=-1)[:, :, None]
            + jnp.sum(dst ** 2, axis=-1)[:, None, :]
            - 2.0 * jnp.matmul(src, jnp.transpose(dst, (0, 2, 1))))


def _index_points(points, idx):
    Bn = points.shape[0]
    batch = jnp.arange(Bn).reshape((Bn,) + (1,) * (idx.ndim - 1))
    return points[batch, idx]


def _fps(xyz, npoint):
    xyz = jax.lax.stop_gradient(xyz)
    Bn, N, _ = xyz.shape
    def step(state, _):
        distance, farthest = state
        centroid = jnp.take_along_axis(xyz, farthest[:, None, None], axis=1)
        dist = jnp.sum((xyz - centroid) ** 2, axis=-1)
        distance = jnp.minimum(distance, dist)
        new_far = jnp.argmax(distance, axis=-1).astype(jnp.int32)
        return (distance, new_far), farthest
    init = (jnp.full((Bn, N), 1e10, dtype=jnp.float32), jnp.zeros((Bn,), dtype=jnp.int32))
    _, cent = jax.lax.scan(step, init, None, length=npoint)
    return jnp.transpose(cent, (1, 0))


def _knn(nsample, xyz, new_xyz):
    sqr = _sqdist(new_xyz, xyz)
    _, idx = jax.lax.top_k(-jax.lax.stop_gradient(sqr), nsample)
    return idx


def _sa(xyz, points, npoint, nsample, layer_params):
    xyz_t = jnp.transpose(xyz, (0, 2, 1))
    pts_t = jnp.transpose(points, (0, 2, 1))
    fps_idx = _fps(xyz_t, npoint)
    new_xyz = _index_points(xyz_t, fps_idx)
    idx = _knn(nsample, xyz_t, new_xyz)
    grouped_xyz = _index_points(xyz_t, idx) - new_xyz[:, :, None, :]
    grouped_pts = _index_points(pts_t, idx)
    new_points = jnp.concatenate([grouped_xyz, grouped_pts], axis=-1)
    x = jnp.transpose(new_points, (0, 3, 2, 1))
    for w, b in layer_params:
        x = jnp.einsum('oc,bcks->boks', w, x) + b[None, :, None, None]
        mean = jnp.mean(x, axis=(2, 3), keepdims=True)
        var = jnp.var(x, axis=(2, 3), keepdims=True)
        x = (x - mean) / jnp.sqrt(var + 1e-5)
        x = jax.nn.relu(x)
    feat = jnp.max(x, axis=2)
    return jnp.transpose(new_xyz, (0, 2, 1)), feat


def _copy_kernel(x_ref, o_ref):
    o_ref[...] = x_ref[...]


def _pl_copy(x):
    return pl.pallas_call(
        _copy_kernel,
        out_shape=jax.ShapeDtypeStruct(x.shape, x.dtype),
    )(x)


def kernel(pc, feature, params):
    pc1, f1 = _sa(pc, feature, _NPOINT // 8, 32, params[0])
    pc2, f2 = _sa(pc1, f1, _NPOINT // 16, 24, params[1])
    pc3, f3 = _sa(pc2, f2, _NPOINT // 32, 16, params[2])
    f3 = _pl_copy(f3)
    return ((pc, pc1, pc2, pc3), f3)


# speedup vs baseline: 1.6064x; 1.6058x over previous
import functools

import jax
import jax.numpy as jnp
from jax.experimental import pallas as pl
from jax.experimental.pallas import tpu as pltpu

_NPOINT = 8192


def _fps_body(xyz_ref, cent_ref, dist_ref, *, nsteps):
    x = xyz_ref[0]
    y = xyz_ref[1]
    z = xyz_ref[2]
    Bn, N = x.shape
    S = cent_ref.shape[2]
    iota = jax.lax.broadcasted_iota(jnp.int32, (Bn, N), 1)
    iota_s = jax.lax.broadcasted_iota(jnp.int32, (Bn, S), 1)
    dist_ref[...] = jnp.full((Bn, N), 1e10, jnp.float32)

    def body(i, far):
        mask = iota == far
        cx = jnp.sum(jnp.where(mask, x, 0.0), axis=1, keepdims=True)
        cy = jnp.sum(jnp.where(mask, y, 0.0), axis=1, keepdims=True)
        cz = jnp.sum(jnp.where(mask, z, 0.0), axis=1, keepdims=True)
        sel = iota_s == i
        cent_ref[0] = jnp.where(sel, cx, cent_ref[0])
        cent_ref[1] = jnp.where(sel, cy, cent_ref[1])
        cent_ref[2] = jnp.where(sel, cz, cent_ref[2])
        dx = x - cx
        dy = y - cy
        dz = z - cz
        dist = (dx * dx + dy * dy) + (dz * dz)
        d = jnp.minimum(dist_ref[...], dist)
        dist_ref[...] = d
        m = jnp.max(d, axis=1, keepdims=True)
        nf = jnp.min(jnp.where(d == m, iota, N), axis=1, keepdims=True)
        return nf

    jax.lax.fori_loop(0, nsteps, body, jnp.zeros((Bn, 1), jnp.int32))


def _fps_pallas(xyz3, npoint, interpret=False):
    _, Bn, N = xyz3.shape
    return pl.pallas_call(
        functools.partial(_fps_body, nsteps=npoint),
        out_shape=jax.ShapeDtypeStruct((3, Bn, npoint), jnp.float32),
        scratch_shapes=[pltpu.VMEM((Bn, N), jnp.float32)],
        interpret=interpret,
    )(xyz3)


def _sqdist(src, dst):
    return (jnp.sum(src ** 2, axis=-1)[:, :, None]
            + jnp.sum(dst ** 2, axis=-1)[:, None, :]
            - 2.0 * jnp.matmul(src, jnp.transpose(dst, (0, 2, 1))))


def _index_points(points, idx):
    Bn = points.shape[0]
    batch = jnp.arange(Bn).reshape((Bn,) + (1,) * (idx.ndim - 1))
    return points[batch, idx]


def _knn(nsample, xyz, new_xyz):
    sqr = _sqdist(new_xyz, xyz)
    _, idx = jax.lax.top_k(-jax.lax.stop_gradient(sqr), nsample)
    return idx


def _sa(xyz3, points, npoint, nsample, layer_params, interpret=False):
    new3 = _fps_pallas(xyz3, npoint, interpret=interpret)
    xyz_t = jnp.transpose(xyz3, (1, 2, 0))
    pts_t = jnp.transpose(points, (0, 2, 1))
    new_xyz = jnp.transpose(new3, (1, 2, 0))
    idx = _knn(nsample, xyz_t, new_xyz)
    grouped_xyz = _index_points(xyz_t, idx) - new_xyz[:, :, None, :]
    grouped_pts = _index_points(pts_t, idx)
    new_points = jnp.concatenate([grouped_xyz, grouped_pts], axis=-1)
    x = jnp.transpose(new_points, (0, 3, 2, 1))
    for w, b in layer_params:
        x = jnp.einsum('oc,bcks->boks', w, x) + b[None, :, None, None]
        mean = jnp.mean(x, axis=(2, 3), keepdims=True)
        var = jnp.var(x, axis=(2, 3), keepdims=True)
        x = (x - mean) / jnp.sqrt(var + 1e-5)
        x = jax.nn.relu(x)
    feat = jnp.max(x, axis=2)
    return new3, feat


def _pipeline(pc, feature, params, interpret=False):
    xyz3 = jnp.transpose(pc, (1, 0, 2))
    n1, f1 = _sa(xyz3, feature, _NPOINT // 8, 32, params[0], interpret)
    n2, f2 = _sa(n1, f1, _NPOINT // 16, 24, params[1], interpret)
    n3, f3 = _sa(n2, f2, _NPOINT // 32, 16, params[2], interpret)
    pc1 = jnp.transpose(n1, (1, 0, 2))
    pc2 = jnp.transpose(n2, (1, 0, 2))
    pc3 = jnp.transpose(n3, (1, 0, 2))
    return ((pc, pc1, pc2, pc3), f3)


def kernel(pc, feature, params):
    return _pipeline(pc, feature, params)


# speedup vs baseline: 4.6849x; 2.9164x over previous
import functools

import jax
import jax.numpy as jnp
from jax.experimental import pallas as pl
from jax.experimental.pallas import tpu as pltpu

_NPOINT = 8192
_INF = 3e38


def _fps_body(xyz_ref, cent_ref, dist_ref, *, nsteps):
    x = xyz_ref[0]
    y = xyz_ref[1]
    z = xyz_ref[2]
    Bn, N = x.shape
    S = cent_ref.shape[2]
    iota = jax.lax.broadcasted_iota(jnp.int32, (Bn, N), 1)
    iota_s = jax.lax.broadcasted_iota(jnp.int32, (Bn, S), 1)
    dist_ref[...] = jnp.full((Bn, N), 1e10, jnp.float32)

    def body(i, far):
        mask = iota == far
        cx = jnp.sum(jnp.where(mask, x, 0.0), axis=1, keepdims=True)
        cy = jnp.sum(jnp.where(mask, y, 0.0), axis=1, keepdims=True)
        cz = jnp.sum(jnp.where(mask, z, 0.0), axis=1, keepdims=True)
        sel = iota_s == i
        cent_ref[0] = jnp.where(sel, cx, cent_ref[0])
        cent_ref[1] = jnp.where(sel, cy, cent_ref[1])
        cent_ref[2] = jnp.where(sel, cz, cent_ref[2])
        dx = x - cx
        dy = y - cy
        dz = z - cz
        dist = (dx * dx + dy * dy) + (dz * dz)
        d = jnp.minimum(dist_ref[...], dist)
        dist_ref[...] = d
        m = jnp.max(d, axis=1, keepdims=True)
        nf = jnp.min(jnp.where(d == m, iota, N), axis=1, keepdims=True)
        return nf

    jax.lax.fori_loop(0, nsteps, body, jnp.zeros((Bn, 1), jnp.int32))


def _fps_pallas(xyz3, npoint, interpret=False):
    _, Bn, N = xyz3.shape
    return pl.pallas_call(
        functools.partial(_fps_body, nsteps=npoint),
        out_shape=jax.ShapeDtypeStruct((3, Bn, npoint), jnp.float32),
        scratch_shapes=[pltpu.VMEM((Bn, N), jnp.float32)],
        interpret=interpret,
    )(xyz3)


def _topk_body(d0_ref, idx_ref, d_ref, *, K, N):
    Sb = d0_ref.shape[1]
    d_ref[...] = d0_ref[0]
    iota = jax.lax.broadcasted_iota(jnp.int32, (Sb, N), 1)
    kiota = jax.lax.broadcasted_iota(jnp.int32, (Sb, K), 1)
    acc = jnp.zeros((Sb, K), jnp.int32)
    for k in range(K):
        d = d_ref[...]
        m = jnp.min(d, axis=1, keepdims=True)
        j = jnp.min(jnp.where(d == m, iota, N), axis=1, keepdims=True)
        acc = jnp.where(kiota == k, j, acc)
        d_ref[...] = jnp.where(iota == j, _INF, d)
    idx_ref[0] = acc


def _knn_pallas(xyz_t, new_xyz, K, Sb, interpret=False):
    B, N, _ = xyz_t.shape
    S = new_xyz.shape[1]
    sqr = (jnp.sum(new_xyz ** 2, axis=-1)[:, :, None]
           + jnp.sum(xyz_t ** 2, axis=-1)[:, None, :]
           - 2.0 * jnp.matmul(new_xyz, jnp.transpose(xyz_t, (0, 2, 1))))
    grid = (B, S // Sb)
    return pl.pallas_call(
        functools.partial(_topk_body, K=K, N=N),
        grid=grid,
        in_specs=[
            pl.BlockSpec((1, Sb, N), lambda b, i: (b, i, 0)),
        ],
        out_specs=pl.BlockSpec((1, Sb, K), lambda b, i: (b, i, 0)),
        out_shape=jax.ShapeDtypeStruct((B, S, K), jnp.int32),
        scratch_shapes=[pltpu.VMEM((Sb, N), jnp.float32)],
        interpret=interpret,
    )(sqr)


def _dot_f32(a, b):
    return jax.lax.dot_general(
        a.astype(jnp.bfloat16), b.astype(jnp.bfloat16),
        (((1,), (0,)), ((), ())),
        preferred_element_type=jnp.float32)

def _lin1_body(g_ref, c_ref, w_ref, b_ref, y_ref, st_ref, *, g_cnt):
    g = g_ref[0]
    ctile = jnp.concatenate([c_ref[0]] * g_cnt, axis=0)
    gfull = jnp.concatenate([g[:, 0:3] - ctile, g[:, 3:]], axis=1)
    y = _dot_f32(gfull, w_ref[...]) + b_ref[...]
    y_ref[0] = y
    st = jnp.stack([jnp.sum(y, axis=0), jnp.sum(y * y, axis=0)])

    @pl.when(pl.program_id(1) == 0)
    def _():
        st_ref[0] = st

    @pl.when(pl.program_id(1) != 0)
    def _():
        st_ref[0] = st_ref[0] + st


def _normlin_body(y_ref, s_ref, w_ref, b_ref, o_ref, st_ref, *, M):
    s = s_ref[0]
    z = jax.nn.relu((y_ref[0] - s[0:1]) * s[1:2])
    o = _dot_f32(z, w_ref[...]) + b_ref[...]
    o_ref[0] = o
    st = jnp.stack([jnp.sum(o, axis=0), jnp.sum(o * o, axis=0)])

    @pl.when(pl.program_id(1) == 0)
    def _():
        st_ref[0] = st

    @pl.when(pl.program_id(1) != 0)
    def _():
        st_ref[0] = st_ref[0] + st


def _normpool_body(y_ref, s_ref, o_ref, *, M, S, g_cnt):
    s = s_ref[0]
    z = jax.nn.relu((y_ref[0] - s[0:1]) * s[1:2])
    m = z[0:S]
    for g in range(1, g_cnt):
        m = jnp.maximum(m, z[g * S:(g + 1) * S])

    @pl.when(pl.program_id(1) == 0)
    def _():
        o_ref[0] = m

    @pl.when(pl.program_id(1) != 0)
    def _():
        o_ref[0] = jnp.maximum(o_ref[0], m)


def _mlp_pool(G, new_xyz, layer_params, S, K, nb, interpret=False):
    B, M, Cin = G.shape
    Mb = M // nb
    g_cnt = Mb // S
    (w1, b1), (w2, b2), (w3, b3) = layer_params
    C1 = w1.shape[0]
    C2 = w2.shape[0]
    C3 = w3.shape[0]
    w1t = jnp.transpose(w1)
    w2t = jnp.transpose(w2)
    w3t = jnp.transpose(w3)
    b1r = b1[None, :]
    b2r = b2[None, :]
    b3r = b3[None, :]
    grid = (B, nb)

    y1, st1 = pl.pallas_call(
        functools.partial(_lin1_body, g_cnt=g_cnt),
        grid=grid,
        in_specs=[
            pl.BlockSpec((1, Mb, Cin), lambda b, i: (b, i, 0)),
            pl.BlockSpec((1, S, 3), lambda b, i: (b, 0, 0)),
            pl.BlockSpec((Cin, C1), lambda b, i: (0, 0)),
            pl.BlockSpec((1, C1), lambda b, i: (0, 0)),
        ],
        out_specs=[
            pl.BlockSpec((1, Mb, C1), lambda b, i: (b, i, 0)),
            pl.BlockSpec((1, 2, C1), lambda b, i: (b, 0, 0)),
        ],
        out_shape=[
            jax.ShapeDtypeStruct((B, M, C1), jnp.float32),
            jax.ShapeDtypeStruct((B, 2, C1), jnp.float32),
        ],
        interpret=interpret,
    )(G, new_xyz, w1t, b1r)

    def norm_consts(st):
        mean = st[:, 0:1] / M
        var = st[:, 1:2] / M - mean * mean
        return jnp.concatenate([mean, 1.0 / jnp.sqrt(var + 1e-5)], axis=1)

    def norm_lin(y, st, wt, br, Cp, Cn):
        return pl.pallas_call(
            functools.partial(_normlin_body, M=float(M)),
            grid=grid,
            in_specs=[
                pl.BlockSpec((1, Mb, Cp), lambda b, i: (b, i, 0)),
                pl.BlockSpec((1, 2, Cp), lambda b, i: (b, 0, 0)),
                pl.BlockSpec((Cp, Cn), lambda b, i: (0, 0)),
                pl.BlockSpec((1, Cn), lambda b, i: (0, 0)),
            ],
            out_specs=[
                pl.BlockSpec((1, Mb, Cn), lambda b, i: (b, i, 0)),
                pl.BlockSpec((1, 2, Cn), lambda b, i: (b, 0, 0)),
            ],
            out_shape=[
                jax.ShapeDtypeStruct((B, M, Cn), jnp.float32),
                jax.ShapeDtypeStruct((B, 2, Cn), jnp.float32),
            ],
            interpret=interpret,
        )(y, norm_consts(st), wt, br)

    y2, st2 = norm_lin(y1, st1, w2t, b2r, C1, C2)
    y3, st3 = norm_lin(y2, st2, w3t, b3r, C2, C3)

    feat = pl.pallas_call(
        functools.partial(_normpool_body, M=float(M), S=S, g_cnt=g_cnt),
        grid=grid,
        in_specs=[
            pl.BlockSpec((1, Mb, C3), lambda b, i: (b, i, 0)),
            pl.BlockSpec((1, 2, C3), lambda b, i: (b, 0, 0)),
        ],
        out_specs=pl.BlockSpec((1, S, C3), lambda b, i: (b, 0, 0)),
        out_shape=jax.ShapeDtypeStruct((B, S, C3), jnp.float32),
        interpret=interpret,
    )(y3, norm_consts(st3))
    return feat


def _sa(xyz3, pts_t, npoint, nsample, layer_params, Sb, nb, interpret=False):
    new3 = _fps_pallas(xyz3, npoint, interpret=interpret)
    xyz_t = jnp.transpose(xyz3, (1, 2, 0))
    new_xyz = jnp.transpose(new3, (1, 2, 0))
    idx = _knn_pallas(xyz_t, new_xyz, nsample, Sb, interpret=interpret)
    B, S, K = idx.shape
    idx_km = jnp.transpose(idx, (0, 2, 1)).reshape(B, K * S)
    P = jnp.concatenate([xyz_t, pts_t], axis=-1)
    G = jnp.take_along_axis(P, idx_km[:, :, None], axis=1)
    feat = _mlp_pool(G, new_xyz, layer_params, S, K, nb, interpret=interpret)
    return new3, feat


def _pipeline(pc, feature, params, interpret=False):
    xyz3 = jnp.transpose(pc, (1, 0, 2))
    pts_t = jnp.transpose(feature, (0, 2, 1))
    n1, f1 = _sa(xyz3, pts_t, _NPOINT // 8, 32, params[0], 256, 8, interpret)
    n2, f2 = _sa(n1, f1, _NPOINT // 16, 24, params[1], 512, 4, interpret)
    n3, f3 = _sa(n2, f2, _NPOINT // 32, 16, params[2], 256, 1, interpret)
    pc1 = jnp.transpose(n1, (1, 0, 2))
    pc2 = jnp.transpose(n2, (1, 0, 2))
    pc3 = jnp.transpose(n3, (1, 0, 2))
    f3 = jnp.transpose(f3, (0, 2, 1))
    return ((pc, pc1, pc2, pc3), f3)


def kernel(pc, feature, params):
    return _pipeline(pc, feature, params)
